# baseline (speedup 1.0000x reference)
import functools

import jax
import jax.numpy as jnp
from jax import lax
from jax.experimental import pallas as pl
from jax.experimental.pallas import tpu as pltpu
from jax.experimental.pallas import tpu_sc as plsc

EPS = 1e-12


def _sc_gather(tok_table, ids, n_rows, d):
    info = plsc.get_sparse_core_info()
    nw = info.num_cores * info.num_subcores
    rows_per_w = n_rows // nw
    chunk = 64

    mesh = plsc.VectorSubcoreMesh(core_axis_name="c", subcore_axis_name="s")

    @functools.partial(
        pl.kernel,
        mesh=mesh,
        out_type=jax.ShapeDtypeStruct((n_rows, d), jnp.float32),
        scratch_types=[
            pltpu.VMEM((rows_per_w,), jnp.int32),
            pltpu.VMEM((chunk, d), jnp.float32),
            pltpu.VMEM((chunk, d), jnp.float32),
            pltpu.SemaphoreType.DMA,
            pltpu.SemaphoreType.DMA,
        ],
    )
    def gather_kernel(table_hbm, idx_hbm, out_hbm, idx_v, r0, r1, s0, s1):
        wid = lax.axis_index("s") * info.num_cores + lax.axis_index("c")
        base = wid * rows_per_w

        pltpu.sync_copy(idx_hbm.at[pl.ds(base, rows_per_w)], idx_v)
        pltpu.async_copy(table_hbm.at[idx_v.at[pl.ds(0, chunk)]], r0, s0)

        @pl.loop(0, rows_per_w, step=2 * chunk)
        def _(c):
            for buf, sem, other_buf, other_sem, off in (
                (r0, s0, r1, s1, chunk),
                (r1, s1, r0, s0, 2 * chunk),
            ):
                nxt = c + off

                @pl.when(nxt < rows_per_w)
                def _():
                    pltpu.async_copy(
                        table_hbm.at[idx_v.at[pl.ds(nxt, chunk)]],
                        other_buf, other_sem)

                pltpu.make_async_copy(table_hbm.at[pl.ds(0, chunk)], buf,
                                      sem).wait()
                pltpu.sync_copy(
                    buf, out_hbm.at[pl.ds(base + nxt - chunk, chunk)])

    return gather_kernel(tok_table, ids)


def _tc_add_ln_band(gathered, pos_table, gamma2, beta2, prev, off, band,
                    n_rows, n_batch, t_len, blk):
    d = pos_table.shape[1]
    jblks = band // blk
    tblks = t_len // blk
    offblk = off // blk

    def body(g_ref, p_ref, gm_ref, bt_ref, *rest):
        o_ref = rest[-1]
        i = pl.program_id(0)
        h = g_ref[...] + p_ref[pl.ds((i % jblks) * blk, blk), :]
        mu = jnp.mean(h, axis=1, keepdims=True)
        hc = h - mu
        var = jnp.mean(hc * hc, axis=1, keepdims=True)
        o_ref[...] = hc * lax.rsqrt(var + EPS) * gm_ref[...] + bt_ref[...]

    in_specs = [
        pl.BlockSpec((blk, d), lambda i: (i, 0)),
        pl.BlockSpec((band, d), lambda i: (off // band, 0)),
        pl.BlockSpec((1, d), lambda i: (0, 0)),
        pl.BlockSpec((1, d), lambda i: (0, 0)),
    ]
    args = [gathered, pos_table, gamma2, beta2]
    aliases = {}
    if prev is not None:
        in_specs.append(pl.BlockSpec((8, 128), lambda i: (0, 0)))
        args.append(prev)
        aliases = {4: 0}

    return pl.pallas_call(
        body,
        grid=(n_batch * jblks,),
        in_specs=in_specs,
        out_specs=pl.BlockSpec(
            (blk, d),
            lambda i: ((i // jblks) * tblks + offblk + i % jblks, 0)),
        out_shape=jax.ShapeDtypeStruct((n_rows, d), jnp.float32),
        input_output_aliases=aliases,
    )(*args)


def kernel(input_ids, tok_table, pos_table, gamma, beta):
    b, t = input_ids.shape
    v, d = tok_table.shape
    n_rows = b * t

    bands = (4096, 2048, 2048)
    gamma2 = gamma.reshape(1, d)
    beta2 = beta.reshape(1, d)

    ids = input_ids.astype(jnp.int32)
    offs = [sum(bands[:k]) for k in range(len(bands))]
    gathered = [
        _sc_gather(tok_table, ids[:, off:off + band].reshape(-1),
                   b * band, d)
        for off, band in zip(offs, bands)
    ]
    out = None
    for g, off, band in zip(gathered, offs, bands):
        out = _tc_add_ln_band(g, pos_table, gamma2, beta2, out, off, band,
                              n_rows, b, t, blk=2048)
    return out.reshape(b, t, d)


# speedup vs baseline: 2.1905x; 2.1905x over previous
import functools

import jax
import jax.numpy as jnp
from jax import lax
from jax.experimental import pallas as pl
from jax.experimental.pallas import tpu as pltpu
from jax.experimental.pallas import tpu_sc as plsc

EPS = 1e-12


def _sc_gather(tok_table, ids, n_rows, d):
    info = plsc.get_sparse_core_info()
    nw = info.num_cores * info.num_subcores
    rows_per_w = n_rows // nw
    chunk = 64

    mesh = plsc.VectorSubcoreMesh(core_axis_name="c", subcore_axis_name="s")

    @functools.partial(
        pl.kernel,
        mesh=mesh,
        out_type=jax.ShapeDtypeStruct((n_rows, d), jnp.float32),
        scratch_types=[
            pltpu.VMEM((rows_per_w,), jnp.int32),
            pltpu.VMEM((chunk, d), jnp.float32),
            pltpu.VMEM((chunk, d), jnp.float32),
            pltpu.SemaphoreType.DMA,
            pltpu.SemaphoreType.DMA,
        ],
    )
    def gather_kernel(table_hbm, idx_hbm, out_hbm, idx_v, r0, r1, s0, s1):
        wid = lax.axis_index("s") * info.num_cores + lax.axis_index("c")
        base = wid * rows_per_w

        pltpu.sync_copy(idx_hbm.at[pl.ds(base, rows_per_w)], idx_v)
        pltpu.async_copy(table_hbm.at[idx_v.at[pl.ds(0, chunk)]], r0, s0)

        @pl.loop(0, rows_per_w, step=2 * chunk)
        def _(c):
            for buf, sem, other_buf, other_sem, off in (
                (r0, s0, r1, s1, chunk),
                (r1, s1, r0, s0, 2 * chunk),
            ):
                nxt = c + off

                @pl.when(nxt < rows_per_w)
                def _():
                    pltpu.async_copy(
                        table_hbm.at[idx_v.at[pl.ds(nxt, chunk)]],
                        other_buf, other_sem)

                pltpu.make_async_copy(table_hbm.at[pl.ds(0, chunk)], buf,
                                      sem).wait()
                pltpu.sync_copy(
                    buf, out_hbm.at[pl.ds(base + nxt - chunk, chunk)])

    return gather_kernel(tok_table, ids)


def _tc_add_ln_band(gathered, pos_table, gamma2, beta2, prev, off, band,
                    n_rows, n_batch, t_len, blk):
    d = pos_table.shape[1]
    jblks = band // blk
    tblks = t_len // blk
    offblk = off // blk

    def body(g_ref, p_ref, gm_ref, bt_ref, *rest):
        o_ref = rest[-1]
        i = pl.program_id(0)
        h = g_ref[...] + p_ref[pl.ds((i % jblks) * blk, blk), :]
        mu = jnp.mean(h, axis=1, keepdims=True)
        hc = h - mu
        var = jnp.mean(hc * hc, axis=1, keepdims=True)
        o_ref[...] = hc * lax.rsqrt(var + EPS) * gm_ref[...] + bt_ref[...]

    in_specs = [
        pl.BlockSpec((blk, d), lambda i: (i, 0)),
        pl.BlockSpec((band, d), lambda i: (off // band, 0)),
        pl.BlockSpec((1, d), lambda i: (0, 0)),
        pl.BlockSpec((1, d), lambda i: (0, 0)),
    ]
    args = [gathered, pos_table, gamma2, beta2]
    aliases = {}
    if prev is not None:
        in_specs.append(pl.BlockSpec((8, 128), lambda i: (0, 0)))
        args.append(prev)
        aliases = {4: 0}

    return pl.pallas_call(
        body,
        grid=(n_batch * jblks,),
        in_specs=in_specs,
        out_specs=pl.BlockSpec(
            (blk, d),
            lambda i: ((i // jblks) * tblks + offblk + i % jblks, 0)),
        out_shape=jax.ShapeDtypeStruct((n_rows, d), jnp.float32),
        input_output_aliases=aliases,
    )(*args)


def kernel(input_ids, tok_table, pos_table, gamma, beta):
    b, t = input_ids.shape
    v, d = tok_table.shape
    n_rows = b * t

    bands = (2048, 2048, 4096)
    gamma2 = gamma.reshape(1, d)
    beta2 = beta.reshape(1, d)

    ids = input_ids.astype(jnp.int32)
    offs = [sum(bands[:k]) for k in range(len(bands))]
    gathered = [
        _sc_gather(tok_table, ids[:, off:off + band].reshape(-1),
                   b * band, d)
        for off, band in zip(offs, bands)
    ]
    out = None
    for g, off, band in zip(gathered, offs, bands):
        out = _tc_add_ln_band(g, pos_table, gamma2, beta2, out, off, band,
                              n_rows, b, t, blk=2048)
    return out.reshape(b, t, d)
